# baseline (speedup 1.0000x reference)
import functools

import jax
import jax.numpy as jnp
from jax import lax
from jax.experimental import pallas as pl
from jax.experimental.pallas import tpu as pltpu
from jax.experimental.pallas import tpu_sc as plsc

B = 32768
H = 768
H2 = 384
E = 64
TB = 4096

NC = 2
NS = 16
NW = NC * NS
RPW = B // NW
CR = 256
NG = CR // 16


def _logits_kernel(x_ref, w1_ref, b1_ref, w2_ref, b2_ref, out_ref):
    x = x_ref[...]
    h = jnp.dot(x, w1_ref[...], preferred_element_type=jnp.float32)
    h = jnp.maximum(h + b1_ref[...], 0.0)
    logits = jnp.dot(h, w2_ref[...], preferred_element_type=jnp.float32)
    out_ref[...] = logits + b2_ref[...]


def _tc_logits(query_repr, W1, b1r, W2, b2r):
    grid = (B // TB,)
    return pl.pallas_call(
        _logits_kernel,
        grid=grid,
        in_specs=[
            pl.BlockSpec((TB, H), lambda i: (i, 0)),
            pl.BlockSpec((H, H2), lambda i: (0, 0)),
            pl.BlockSpec((1, H2), lambda i: (0, 0)),
            pl.BlockSpec((H2, E), lambda i: (0, 0)),
            pl.BlockSpec((1, E), lambda i: (0, 0)),
        ],
        out_specs=pl.BlockSpec((TB, E), lambda i: (i, 0)),
        out_shape=jax.ShapeDtypeStruct((B, E), jnp.float32),
        compiler_params=pltpu.CompilerParams(
            dimension_semantics=("parallel",),
        ),
    )(query_repr, W1, b1r, W2, b2r)


def _scan_experts(in_v, row_idx, e_lo, e_hi):
    m1 = jnp.full((16,), -jnp.inf, jnp.float32)
    m2 = jnp.full((16,), -jnp.inf, jnp.float32)
    i1 = jnp.zeros((16,), jnp.int32)
    i2 = jnp.zeros((16,), jnp.int32)
    for e in range(e_lo, e_hi):
        ve = plsc.load_gather(in_v, [row_idx, jnp.full((16,), e, jnp.int32)])
        gt1 = ve > m1
        gt2 = jnp.logical_and(jnp.logical_not(gt1), ve > m2)
        ei = jnp.full((16,), e, jnp.int32)
        m2 = jnp.where(gt1, m1, jnp.where(gt2, ve, m2))
        i2 = jnp.where(gt1, i1, jnp.where(gt2, ei, i2))
        m1 = jnp.where(gt1, ve, m1)
        i1 = jnp.where(gt1, ei, i1)
    return m1, i1, m2, i2


def _merge_top2(a, b):
    am1, ai1, am2, ai2 = a
    bm1, bi1, bm2, bi2 = b
    a_wins = am1 >= bm1
    m1 = jnp.where(a_wins, am1, bm1)
    i1 = jnp.where(a_wins, ai1, bi1)
    cm = jnp.where(a_wins, bm1, am1)
    ci = jnp.where(a_wins, bi1, ai1)
    sm = jnp.where(a_wins, am2, bm2)
    si = jnp.where(a_wins, ai2, bi2)
    s_wins = jnp.where(a_wins, sm >= cm, sm > cm)
    m2 = jnp.where(s_wins, sm, cm)
    i2 = jnp.where(s_wins, si, ci)
    return m1, i1, m2, i2


def _sc_tail_body(logits_hbm, out_hbm, in_v, out_v):
    wid = lax.axis_index("s") * NC + lax.axis_index("c")
    base = wid * RPW
    lanes = lax.iota(jnp.int32, 16)
    zero16 = jnp.zeros((16,), jnp.float32)
    for chunk in range(RPW // CR):
        cbase = base + chunk * CR
        pltpu.sync_copy(logits_hbm.at[pl.ds(cbase, CR)], in_v)

        def group_body(g, carry):
            row_idx = lanes + g * 16
            sa = _scan_experts(in_v, row_idx, 0, E // 2)
            sb = _scan_experts(in_v, row_idx, E // 2, E)
            m1, i1, m2, i2 = _merge_top2(sa, sb)
            for e in range(E):
                plsc.store_scatter(
                    out_v, [row_idx, jnp.full((16,), e, jnp.int32)], zero16
                )
            e2 = jnp.exp(m2 - m1)
            g1 = 1.0 / (1.0 + e2)
            g2 = e2 * g1
            plsc.store_scatter(out_v, [row_idx, i1], g1)
            plsc.store_scatter(out_v, [row_idx, i2], g2)
            return carry

        lax.fori_loop(0, NG, group_body, 0)
        pltpu.sync_copy(out_v, out_hbm.at[pl.ds(cbase, CR)])


_sc_tail = functools.partial(
    pl.kernel,
    mesh=plsc.VectorSubcoreMesh(core_axis_name="c", subcore_axis_name="s"),
    out_type=jax.ShapeDtypeStruct((B, E), jnp.float32),
    scratch_types=[
        pltpu.VMEM((CR, E), jnp.float32),
        pltpu.VMEM((CR, E), jnp.float32),
    ],
    compiler_params=pltpu.CompilerParams(needs_layout_passes=False),
)(_sc_tail_body)


def kernel(query_repr, W1, b1, W2, b2):
    b1r = b1.reshape(1, H2)
    b2r = b2.reshape(1, E)
    logits = _tc_logits(query_repr, W1, b1r, W2, b2r)
    return _sc_tail(logits)


# speedup vs baseline: 2.1229x; 2.1229x over previous
import jax
import jax.numpy as jnp
from jax.experimental import pallas as pl
from jax.experimental.pallas import tpu as pltpu

B = 32768
H = 768
H2 = 384
E = 64
TB = 4096


def _gating_kernel(x_ref, w1_ref, b1_ref, w2_ref, b2_ref, out_ref):
    x = x_ref[...]
    h = jnp.dot(x, w1_ref[...], preferred_element_type=jnp.float32)
    h = jnp.maximum(h + b1_ref[...], 0.0)
    logits = jnp.dot(h, w2_ref[...], preferred_element_type=jnp.float32)
    logits = logits + b2_ref[...]

    ncol = -jax.lax.broadcasted_iota(jnp.int32, logits.shape, 1).astype(jnp.float32)
    ninf = jnp.float32(-jnp.inf)
    m1 = jnp.max(logits, axis=1, keepdims=True)
    t1 = jnp.where(logits == m1, ncol, ninf)
    i1n = jnp.max(t1, axis=1, keepdims=True)
    is1 = t1 == i1n
    masked = jnp.where(is1, ninf, logits)
    m2 = jnp.max(masked, axis=1, keepdims=True)
    t2 = jnp.where(masked == m2, ncol, ninf)
    i2n = jnp.max(t2, axis=1, keepdims=True)
    is2 = t2 == i2n
    e2 = jnp.exp(m2 - m1)
    g1 = 1.0 / (1.0 + e2)
    g2 = e2 * g1
    out_ref[...] = jnp.where(is1, g1, jnp.where(is2, g2, 0.0))


def kernel(query_repr, W1, b1, W2, b2):
    b1r = b1.reshape(1, H2)
    b2r = b2.reshape(1, E)
    grid = (B // TB,)
    return pl.pallas_call(
        _gating_kernel,
        grid=grid,
        in_specs=[
            pl.BlockSpec((TB, H), lambda i: (i, 0)),
            pl.BlockSpec((H, H2), lambda i: (0, 0)),
            pl.BlockSpec((1, H2), lambda i: (0, 0)),
            pl.BlockSpec((H2, E), lambda i: (0, 0)),
            pl.BlockSpec((1, E), lambda i: (0, 0)),
        ],
        out_specs=pl.BlockSpec((TB, E), lambda i: (i, 0)),
        out_shape=jax.ShapeDtypeStruct((B, E), jnp.float32),
        compiler_params=pltpu.CompilerParams(
            dimension_semantics=("parallel",),
        ),
    )(query_repr, W1, b1r, W2, b2r)
